# baseline (speedup 1.0000x reference)
import functools

import jax
import jax.numpy as jnp
from jax import lax
from jax.experimental import pallas as pl
from jax.experimental.pallas import tpu as pltpu
from jax.experimental.pallas import tpu_sc as plsc

_B = 16384
_D = 128
_NPAIRS = 8
_CH = 128
_BB = 4096

_NBUF = 4
_NSLICE = 2


@functools.cache
def _make_sc_gather(bsz, soff):
    info = plsc.get_sparse_core_info()
    nc, ns = info.num_cores, info.num_subcores
    nw = nc * ns
    rpw = bsz // nw
    ch = min(_CH, rpw)
    nch = rpw // ch
    nk = _NPAIRS * nch
    mesh = plsc.VectorSubcoreMesh(core_axis_name="c", subcore_axis_name="s")

    @functools.partial(
        pl.kernel,
        mesh=mesh,
        out_type=jax.ShapeDtypeStruct((_NPAIRS, bsz, _D), jnp.float32),
        scratch_types=[
            pltpu.VMEM((_NPAIRS, rpw), jnp.int32),
            *[pltpu.VMEM((ch, _D), jnp.float32) for _ in range(_NBUF)],
            pltpu.SemaphoreType.DMA,
            pltpu.SemaphoreType.DMA,
            pltpu.SemaphoreType.DMA,
        ],
    )
    def _sc_gather(e0, e1, e2, e3, i0, i1, i2, i3, i4, i5, i6, i7,
                   out_hbm, idx_slab, *bufs_sems):
        rows = bufs_sems[:_NBUF]
        isem, gsem, wsem = bufs_sems[_NBUF:_NBUF + 3]
        tables = [e0, e1, e2, e3]
        idxs = [i0, i1, i2, i3, i4, i5, i6, i7]
        wid = lax.axis_index("s") * nc + lax.axis_index("c")
        base = wid * rpw
        ih = [pltpu.async_copy(idxs[j].at[pl.ds(soff + base, rpw)],
                               idx_slab.at[j], isem)
              for j in range(_NPAIRS)]
        for h in ih:
            h.wait()

        gh, wh = {}, {}

        def start_g(k):
            j, c = divmod(k, nch)
            gh[k] = pltpu.async_copy(
                tables[j % 4].at[idx_slab.at[j, pl.ds(c * ch, ch)]],
                rows[k % _NBUF], gsem)

        def start_w(k):
            j, c = divmod(k, nch)
            wh[k] = pltpu.async_copy(
                rows[k % _NBUF], out_hbm.at[j, pl.ds(base + c * ch, ch)],
                wsem)

        waited = set()
        for k in range(_NBUF - 1):
            start_g(k)
        for k in range(nk):
            gh[k].wait()
            start_w(k)
            nxt = k + _NBUF - 1
            if nxt < nk:
                free = nxt - _NBUF
                if free >= 0:
                    wh[free].wait()
                    waited.add(free)
                start_g(nxt)
        for k in range(nk):
            if k not in waited:
                wh[k].wait()

    return _sc_gather


def _tc_body(g_ref, w1sd_ref, w1sim_ref, b1_ref, w2_ref, b2_ref, w3_ref,
             b3_ref, out_ref):
    eps2 = 1e-16
    f32 = jnp.float32
    mm = lambda a, b: lax.dot_general(
        a, b, (((1,), (0,)), ((), ())),
        preferred_element_type=f32, precision=lax.Precision.DEFAULT)
    acc = b1_ref[...] + jnp.zeros((_BB, 64), f32)
    for f in range(4):
        s = g_ref[f]
        d = g_ref[4 + f]
        acc = acc + mm(s, w1sd_ref[f]) + mm(d, w1sd_ref[4 + f])
        sd = jnp.sum(s * d, axis=1, keepdims=True)
        ss = jnp.sum(s * s, axis=1, keepdims=True)
        dd = jnp.sum(d * d, axis=1, keepdims=True)
        sim = sd * lax.rsqrt(jnp.maximum(ss, eps2) * jnp.maximum(dd, eps2))
        acc = acc + sim * w1sim_ref[f]
    h1 = jnp.maximum(acc, 0.0)
    h2 = jnp.maximum(mm(h1, w2_ref[...]) + b2_ref[...], 0.0)
    z = mm(h2, w3_ref[...]) + b3_ref[...]
    out_ref[...] = jax.nn.sigmoid(z)


def _tc_head(g, w1sd, w1sim, b1r, w2, b2r, w3, b3r, bsz):
    grid = (bsz // _BB,)
    full = lambda shape: pl.BlockSpec(shape, lambda i: (0,) * len(shape))
    return pl.pallas_call(
        _tc_body,
        grid=grid,
        in_specs=[
            pl.BlockSpec((_NPAIRS, _BB, _D), lambda i: (0, i, 0)),
            full((_NPAIRS, _D, 64)),
            full((4, 1, 64)),
            full((1, 64)),
            full((64, 32)),
            full((1, 32)),
            full((32, 1)),
            full((1, 1)),
        ],
        out_specs=pl.BlockSpec((_BB, 1), lambda i: (i, 0)),
        out_shape=jax.ShapeDtypeStruct((bsz, 1), jnp.float32),
    )(g, w1sd, w1sim, b1r, w2, b2r, w3, b3r)


def kernel(src_f0, dst_f0, emb_f0, src_f1, dst_f1, emb_f1,
           src_f2, dst_f2, emb_f2, src_f3, dst_f3, emb_f3,
           W1, b1, W2, b2, W3, b3):
    idxs = [i.astype(jnp.int32)
            for i in (src_f0, src_f1, src_f2, src_f3,
                      dst_f0, dst_f1, dst_f2, dst_f3)]
    bsz = _B // _NSLICE
    w1sd = W1[:1024].reshape(_NPAIRS, _D, 64)
    w1sim = W1[1024:1028].reshape(4, 1, 64)
    gs = [_make_sc_gather(bsz, s * bsz)(emb_f0, emb_f1, emb_f2, emb_f3, *idxs)
          for s in range(_NSLICE)]
    outs = [_tc_head(g, w1sd, w1sim, b1.reshape(1, 64), W2,
                     b2.reshape(1, 32), W3, b3.reshape(1, 1), bsz)
            for g in gs]
    return jnp.concatenate(outs, axis=0).reshape(_B)


# speedup vs baseline: 1.0011x; 1.0011x over previous
import functools

import jax
import jax.numpy as jnp
from jax import lax
from jax.experimental import pallas as pl
from jax.experimental.pallas import tpu as pltpu
from jax.experimental.pallas import tpu_sc as plsc

_B = 16384
_D = 128
_NPAIRS = 8
_CH = 128
_BB = 2048

_NBUF = 4
_SLICES = (8192, 8192)


@functools.cache
def _make_sc_gather(bsz, soff):
    info = plsc.get_sparse_core_info()
    nc, ns = info.num_cores, info.num_subcores
    nw = nc * ns
    rpw = bsz // nw
    ch = max(c for c in range(8, _CH + 1, 8) if rpw % c == 0)
    nch = rpw // ch
    nk = _NPAIRS * nch
    mesh = plsc.VectorSubcoreMesh(core_axis_name="c", subcore_axis_name="s")

    @functools.partial(
        pl.kernel,
        mesh=mesh,
        out_type=jax.ShapeDtypeStruct((_NPAIRS, bsz, _D), jnp.float32),
        scratch_types=[
            pltpu.VMEM((_NPAIRS, rpw), jnp.int32),
            *[pltpu.VMEM((ch, _D), jnp.float32) for _ in range(_NBUF)],
            pltpu.SemaphoreType.DMA,
            pltpu.SemaphoreType.DMA,
            pltpu.SemaphoreType.DMA,
        ],
    )
    def _sc_gather(e0, e1, e2, e3, i0, i1, i2, i3, i4, i5, i6, i7,
                   out_hbm, idx_slab, *bufs_sems):
        rows = bufs_sems[:_NBUF]
        isem, gsem, wsem = bufs_sems[_NBUF:_NBUF + 3]
        tables = [e0, e1, e2, e3]
        idxs = [i0, i1, i2, i3, i4, i5, i6, i7]
        wid = lax.axis_index("s") * nc + lax.axis_index("c")
        base = wid * rpw
        ih = [pltpu.async_copy(idxs[j].at[pl.ds(soff + base, rpw)],
                               idx_slab.at[j], isem)
              for j in range(_NPAIRS)]
        for h in ih:
            h.wait()

        gh, wh = {}, {}

        def start_g(k):
            j, c = divmod(k, nch)
            gh[k] = pltpu.async_copy(
                tables[j % 4].at[idx_slab.at[j, pl.ds(c * ch, ch)]],
                rows[k % _NBUF], gsem)

        def start_w(k):
            j, c = divmod(k, nch)
            wh[k] = pltpu.async_copy(
                rows[k % _NBUF], out_hbm.at[j, pl.ds(base + c * ch, ch)],
                wsem)

        waited = set()
        for k in range(_NBUF - 1):
            start_g(k)
        for k in range(nk):
            gh[k].wait()
            start_w(k)
            nxt = k + _NBUF - 1
            if nxt < nk:
                free = nxt - _NBUF
                if free >= 0:
                    wh[free].wait()
                    waited.add(free)
                start_g(nxt)
        for k in range(nk):
            if k not in waited:
                wh[k].wait()

    return _sc_gather


def _tc_body(g_ref, w1sd_ref, w1sim_ref, b1_ref, w2_ref, b2_ref, w3_ref,
             b3_ref, out_ref):
    eps2 = 1e-16
    f32 = jnp.float32
    mm = lambda a, b: lax.dot_general(
        a, b, (((1,), (0,)), ((), ())),
        preferred_element_type=f32, precision=lax.Precision.DEFAULT)
    acc = b1_ref[...] + jnp.zeros((_BB, 64), f32)
    for f in range(4):
        s = g_ref[f]
        d = g_ref[4 + f]
        acc = acc + mm(s, w1sd_ref[f]) + mm(d, w1sd_ref[4 + f])
        sd = jnp.sum(s * d, axis=1, keepdims=True)
        ss = jnp.sum(s * s, axis=1, keepdims=True)
        dd = jnp.sum(d * d, axis=1, keepdims=True)
        sim = sd * lax.rsqrt(jnp.maximum(ss, eps2) * jnp.maximum(dd, eps2))
        acc = acc + sim * w1sim_ref[f]
    h1 = jnp.maximum(acc, 0.0)
    h2 = jnp.maximum(mm(h1, w2_ref[...]) + b2_ref[...], 0.0)
    z = mm(h2, w3_ref[...]) + b3_ref[...]
    out_ref[...] = jax.nn.sigmoid(z)


def _tc_head(g, w1sd, w1sim, b1r, w2, b2r, w3, b3r, bsz):
    grid = (bsz // _BB,)
    full = lambda shape: pl.BlockSpec(shape, lambda i: (0,) * len(shape))
    return pl.pallas_call(
        _tc_body,
        grid=grid,
        in_specs=[
            pl.BlockSpec((_NPAIRS, _BB, _D), lambda i: (0, i, 0)),
            full((_NPAIRS, _D, 64)),
            full((4, 1, 64)),
            full((1, 64)),
            full((64, 32)),
            full((1, 32)),
            full((32, 1)),
            full((1, 1)),
        ],
        out_specs=pl.BlockSpec((_BB, 1), lambda i: (i, 0)),
        out_shape=jax.ShapeDtypeStruct((bsz, 1), jnp.float32),
    )(g, w1sd, w1sim, b1r, w2, b2r, w3, b3r)


def kernel(src_f0, dst_f0, emb_f0, src_f1, dst_f1, emb_f1,
           src_f2, dst_f2, emb_f2, src_f3, dst_f3, emb_f3,
           W1, b1, W2, b2, W3, b3):
    idxs = [i.astype(jnp.int32)
            for i in (src_f0, src_f1, src_f2, src_f3,
                      dst_f0, dst_f1, dst_f2, dst_f3)]
    w1sd = W1[:1024].reshape(_NPAIRS, _D, 64)
    w1sim = W1[1024:1028].reshape(4, 1, 64)
    offs = [sum(_SLICES[:s]) for s in range(len(_SLICES))]
    gs = [_make_sc_gather(bsz, soff)(emb_f0, emb_f1, emb_f2, emb_f3, *idxs)
          for bsz, soff in zip(_SLICES, offs)]
    outs = [_tc_head(g, w1sd, w1sim, b1.reshape(1, 64), W2,
                     b2.reshape(1, 32), W3, b3.reshape(1, 1), bsz)
            for g, bsz in zip(gs, _SLICES)]
    return jnp.concatenate(outs, axis=0).reshape(_B)
